# baseline (speedup 1.0000x reference)
import functools

import jax
import jax.numpy as jnp
from jax import lax
from jax.experimental import pallas as pl
from jax.experimental.pallas import tpu as pltpu
from jax.experimental.pallas import tpu_sc as plsc

L = 16


def _build_sc_kernel(B, N, G, mb, CHUNK):
    VECS = CHUNK // L
    NCHUNK = N // CHUNK
    C3 = B * G * 3

    mesh = plsc.VectorSubcoreMesh(core_axis_name="c", subcore_axis_name="s")

    @functools.partial(
        pl.kernel,
        out_type=[
            jax.ShapeDtypeStruct((B * mb * N * 5,), jnp.float32),
            jax.ShapeDtypeStruct((B * mb * N,), jnp.int32),
        ],
        mesh=mesh,
        compiler_params=pltpu.CompilerParams(needs_layout_passes=False),
        scratch_types=[
            pltpu.VMEM((C3,), jnp.float32),
            pltpu.VMEM((CHUNK,), jnp.int32),
            pltpu.VMEM((3 * CHUNK,), jnp.float32),
            pltpu.VMEM((CHUNK,), jnp.float32),
            pltpu.VMEM((CHUNK,), jnp.float32),
            pltpu.VMEM((5 * CHUNK,), jnp.float32),
            pltpu.VMEM((5 * CHUNK,), jnp.float32),
            pltpu.VMEM((CHUNK,), jnp.int32),
            pltpu.VMEM((CHUNK,), jnp.int32),
            pltpu.VMEM((L,), jnp.int32),
        ],
    )
    def sc_fn(xyz_hbm, cent_hbm, feats_hbm, idx_hbm, shift_hbm,
              out_hbm, idx2_hbm,
              ctab, idx_v, xyz_v, f0_v, f1_v, stg0, stg1, i2a, i2b, sh_v):
        b = lax.axis_index("s") * 2 + lax.axis_index("c")

        pltpu.sync_copy(cent_hbm, ctab)
        pltpu.sync_copy(shift_hbm, sh_v)
        shiftv = sh_v[...]
        c0v = shiftv + b * G
        c1v = c0v + G

        def chunk_body(g, carry):
            n0 = g * CHUNK
            p = b * N + n0
            pltpu.sync_copy(idx_hbm.at[pl.ds(p, CHUNK)], idx_v)
            pltpu.sync_copy(xyz_hbm.at[pl.ds(p * 3, 3 * CHUNK)], xyz_v)
            r0 = (mb * b) * N + n0
            pltpu.sync_copy(feats_hbm.at[pl.ds(r0, CHUNK)], f0_v)
            pltpu.sync_copy(feats_hbm.at[pl.ds(r0 + N, CHUNK)], f1_v)

            for v in range(VECS):
                idxv = idx_v[pl.ds(v * L, L)]
                ci = idxv * 3
                cx = plsc.load_gather(ctab, [ci])
                cy = plsc.load_gather(ctab, [ci + 1])
                cz = plsc.load_gather(ctab, [ci + 2])
                xidx = jnp.arange(L, dtype=jnp.int32) * 3 + (3 * L * v)
                px = plsc.load_gather(xyz_v, [xidx])
                py = plsc.load_gather(xyz_v, [xidx + 1])
                pz = plsc.load_gather(xyz_v, [xidx + 2])
                dx = px - cx
                dy = py - cy
                dz = pz - cz
                d2 = dx * dx + dy * dy + dz * dz
                ii = lax.bitcast_convert_type(d2, jnp.int32)
                ii = jnp.int32(0x5F3759DF) - lax.shift_right_logical(ii, 1)
                y = lax.bitcast_convert_type(ii, jnp.float32)
                xh = d2 * jnp.float32(0.5)
                for _ in range(3):
                    y = y * (jnp.float32(1.5) - xh * y * y)
                dist = d2 * y
                inv = jnp.float32(1.0) / (dist + jnp.float32(1e-8))
                nx = dx * inv
                ny = dy * inv
                nz = dz * inv
                f0 = f0_v[pl.ds(v * L, L)]
                f1 = f1_v[pl.ds(v * L, L)]
                s0 = jnp.arange(L, dtype=jnp.int32) * 5 + (5 * L * v)
                plsc.store_scatter(stg0, [s0], f0)
                plsc.store_scatter(stg0, [s0 + 1], nx)
                plsc.store_scatter(stg0, [s0 + 2], ny)
                plsc.store_scatter(stg0, [s0 + 3], nz)
                plsc.store_scatter(stg0, [s0 + 4], dist)
                plsc.store_scatter(stg1, [s0], f1)
                plsc.store_scatter(stg1, [s0 + 1], nx)
                plsc.store_scatter(stg1, [s0 + 2], ny)
                plsc.store_scatter(stg1, [s0 + 3], nz)
                plsc.store_scatter(stg1, [s0 + 4], dist)
                i2a[pl.ds(v * L, L)] = idxv + c0v
                i2b[pl.ds(v * L, L)] = idxv + c1v

            o0 = r0 * 5
            pltpu.sync_copy(stg0, out_hbm.at[pl.ds(o0, 5 * CHUNK)])
            pltpu.sync_copy(stg1, out_hbm.at[pl.ds(o0 + N * 5, 5 * CHUNK)])
            pltpu.sync_copy(i2a, idx2_hbm.at[pl.ds(r0, CHUNK)])
            pltpu.sync_copy(i2b, idx2_hbm.at[pl.ds(r0 + N, CHUNK)])
            return carry

        lax.fori_loop(0, NCHUNK, chunk_body, 0)

    return sc_fn


def kernel(xyz, centers, feats, idx, mask_batch):
    B, N, _ = xyz.shape
    G = centers.shape[1]
    mb = feats.shape[0] // B
    shift = jnp.full((L,), jnp.asarray(mask_batch, jnp.int32) - mb,
                     dtype=jnp.int32)
    sc_fn = _build_sc_kernel(B, N, G, mb, CHUNK=512)
    out_flat, idx2 = sc_fn(
        xyz.reshape(-1), centers.reshape(-1), feats.reshape(-1), idx, shift)
    return out_flat.reshape(B * mb, N, 5), idx2
